# baseline (speedup 1.0000x reference)
import functools
import math

import jax
import jax.numpy as jnp
from jax import lax
from jax.experimental import pallas as pl
from jax.experimental.pallas import tpu as pltpu
from jax.experimental.pallas import tpu_sc as plsc

_NB = 32
_H = 16
_N = 2048
_DW = 4480
_D8W = 4352
_LOG_DENOM = math.log(128 / 8)


def _diag_values(table_ref):
    j = jax.lax.broadcasted_iota(jnp.int32, (1, _DW), 1)
    rel = j - (_N - 1)
    neg = -rel
    res = jnp.where(neg < 0, _NB // 2, 0).astype(jnp.int32)
    na = jnp.abs(neg)
    is_small = na < 8
    n_safe = jnp.maximum(na, 1).astype(jnp.float32)
    vil = 8 + (jnp.log(n_safe / 8) / _LOG_DENOM * 8).astype(jnp.int32)
    vil = jnp.minimum(vil, 15)
    bucket = res + jnp.where(is_small, na, vil)
    acc = jnp.zeros((1, _DW), jnp.float32)
    for b in range(_NB):
        acc = jnp.where(bucket == b, table_ref[0, 0, b], acc)
    return acc


def _diag8_body(table_ref, out_ref):
    diag = _diag_values(table_ref)
    rows = jnp.concatenate(
        [pltpu.roll(diag, (_DW - r) % _DW, 1) for r in range(8)], axis=0)
    out_ref[0] = rows[:, :_D8W]


@jax.jit
def _diag8_tc(table_t):
    return pl.pallas_call(
        _diag8_body,
        grid=(_H,),
        in_specs=[pl.BlockSpec((1, 1, _NB), lambda h: (h, 0, 0))],
        out_specs=pl.BlockSpec((1, 8, _D8W), lambda h: (h, 0, 0)),
        out_shape=jax.ShapeDtypeStruct((_H, 8, _D8W), jnp.float32),
    )(table_t)


_RING = 8


def _sc_expand_body(diag8_hbm, out_hbm, d8_v, sem):
    h = lax.axis_index("s")
    half = lax.axis_index("c")
    pltpu.sync_copy(diag8_hbm.at[h], d8_v)
    q0 = half * (_N // 2)

    def row_copy(i):
        q = q0 + i
        s0 = (_N - 1) - q
        r = lax.rem(s0, 8)
        base = s0 - r
        off = pl.multiple_of(r * _D8W + base, 8)
        return pltpu.make_async_copy(
            d8_v.at[pl.ds(off, _N)], out_hbm.at[h, q, :], sem)

    def body(i, carry):
        row_copy(i).start()

        @pl.when(i >= _RING)
        def _():
            row_copy(i - _RING).wait()

        return carry

    lax.fori_loop(0, _N // 2, body, 0)

    def drain(i, carry):
        row_copy(i).wait()
        return carry

    lax.fori_loop(_N // 2 - _RING, _N // 2, drain, 0)


_sc_expand = functools.partial(
    pl.kernel,
    mesh=plsc.VectorSubcoreMesh(core_axis_name="c", subcore_axis_name="s"),
    out_type=jax.ShapeDtypeStruct((_H, _N, _N), jnp.float32),
    scratch_types=[
        pltpu.VMEM((8 * _D8W,), jnp.float32),
        pltpu.SemaphoreType.DMA,
    ],
    compiler_params=pltpu.CompilerParams(use_tc_tiling_on_sc=False),
)(_sc_expand_body)


def kernel(n, rel_bias_table):
    del n
    table_t = rel_bias_table.T.reshape(_H, 1, _NB)
    diag8 = _diag8_tc(table_t).reshape(_H, 8 * _D8W)
    return _sc_expand(diag8)
